# Initial kernel scaffold; baseline (speedup 1.0000x reference)
import jax
import jax.numpy as jnp
from jax.experimental import pallas as pl


def kernel(x, edge_index, edge_weight, W1, b1, W2, b2):
    raise NotImplementedError("write your pallas kernel here")


# speedup vs baseline: 3.8603x; 3.8603x over previous
import functools

import jax
import jax.numpy as jnp
from jax import lax
from jax.experimental import pallas as pl
from jax.experimental.pallas import tpu as pltpu
from jax.experimental.pallas import tpu_sc as plsc

N_NODES = 10000
NP = 10240
D = 128
NC = 2
NS = 16
L = 16
NW = NC * NS
CHUNK = 128


def _make_agg(n_chunks):
  rows_per_tile = NP // NS
  nfull = rows_per_tile // CHUNK

  mesh = plsc.VectorSubcoreMesh(
      core_axis_name="c", subcore_axis_name="s", num_cores=NC, num_subcores=NS)

  @functools.partial(
      pl.kernel,
      out_type=jax.ShapeDtypeStruct((NC, NP, D), jnp.float32),
      mesh=mesh,
      scratch_types=[
          pltpu.VMEM((n_chunks, CHUNK), jnp.int32),
          pltpu.VMEM((n_chunks, CHUNK), jnp.int32),
          pltpu.VMEM((n_chunks, CHUNK), jnp.float32),
          pltpu.VMEM((CHUNK, D), jnp.float32),
          pltpu.VMEM_SHARED((NP, D), jnp.float32),
          pltpu.SemaphoreType.DMA,
      ],
  )
  def agg(x_hbm, src_hbm, dst_hbm, w_hbm, out_hbm,
          src_v, dst_v, w_v, rows_v, acc, sem):
    c = lax.axis_index("c")
    s = lax.axis_index("s")
    wid = s * NC + c

    pltpu.sync_copy(src_hbm.at[wid], src_v)
    pltpu.sync_copy(dst_hbm.at[wid], dst_v)
    pltpu.sync_copy(w_hbm.at[wid], w_v)

    zero = jnp.zeros((L,), jnp.float32)

    def zbody(e, _):
      for k in range(D // L):
        rows_v[e, pl.ds(k * L, L)] = zero
      return 0
    lax.fori_loop(0, CHUNK, zbody, 0)

    base = s * rows_per_tile
    for k in range(nfull):
      pltpu.sync_copy(rows_v, acc.at[pl.ds(base + k * CHUNK, CHUNK)])
    plsc.subcore_barrier()

    def chunk_body(j, _):
      pltpu.async_copy(x_hbm.at[src_v.at[j]], rows_v, sem).wait()

      def gbody(g, _):
        wv = w_v[j, pl.ds(g * L, L)]
        for e in range(L):
          we = wv[e]
          row = g * L + e
          for k in range(D // L):
            sl = pl.ds(k * L, L)
            rows_v[row, sl] = rows_v[row, sl] * we
        return 0
      lax.fori_loop(0, CHUNK // L, gbody, 0)

      pltpu.sync_copy(rows_v, acc.at[dst_v.at[j]], add=True)
      return 0
    lax.fori_loop(0, n_chunks, chunk_body, 0)
    plsc.subcore_barrier()

    for k in range(nfull):
      sl = pl.ds(base + k * CHUNK, CHUNK)
      pltpu.sync_copy(acc.at[sl], out_hbm.at[c].at[sl])

  return agg


BN = 1000


def _mm_tanh_body(p0_ref, p1_ref, w_ref, b_ref, o_ref):
  acc = p0_ref[...] + p1_ref[...]
  o_ref[...] = jnp.tanh(
      jnp.dot(acc, w_ref[...], preferred_element_type=jnp.float32) + b_ref[...])


def _mm_tanh(partials, W, b):
  return pl.pallas_call(
      _mm_tanh_body,
      grid=(N_NODES // BN,),
      in_specs=[
          pl.BlockSpec((BN, D), lambda i: (i, 0)),
          pl.BlockSpec((BN, D), lambda i: (i, 0)),
          pl.BlockSpec((D, D), lambda i: (0, 0)),
          pl.BlockSpec((1, D), lambda i: (0, 0)),
      ],
      out_specs=pl.BlockSpec((BN, D), lambda i: (i, 0)),
      out_shape=jax.ShapeDtypeStruct((N_NODES, D), jnp.float32),
  )(partials[0], partials[1], W, b.reshape(1, D))


def _mm_tanh_mean_body(p0_ref, p1_ref, w_ref, b_ref, o_ref):
  i = pl.program_id(0)
  t = jnp.tanh(
      jnp.dot(p0_ref[...] + p1_ref[...], w_ref[...],
              preferred_element_type=jnp.float32) + b_ref[...])
  part = jnp.sum(t, axis=0, keepdims=True) * (1.0 / N_NODES)

  @pl.when(i == 0)
  def _():
    o_ref[...] = part

  @pl.when(i != 0)
  def _():
    o_ref[...] = o_ref[...] + part


def _mm_tanh_mean(partials, W, b):
  return pl.pallas_call(
      _mm_tanh_mean_body,
      grid=(N_NODES // BN,),
      in_specs=[
          pl.BlockSpec((BN, D), lambda i: (i, 0)),
          pl.BlockSpec((BN, D), lambda i: (i, 0)),
          pl.BlockSpec((D, D), lambda i: (0, 0)),
          pl.BlockSpec((1, D), lambda i: (0, 0)),
      ],
      out_specs=pl.BlockSpec((1, D), lambda i: (0, 0)),
      out_shape=jax.ShapeDtypeStruct((1, D), jnp.float32),
  )(partials[0], partials[1], W, b.reshape(1, D))


@jax.jit
def kernel(x, edge_index, edge_weight, W1, b1, W2, b2):
  e = edge_index.shape[1]
  per_tile = -(-e // (NW * CHUNK)) * CHUNK
  n_chunks = per_tile // CHUNK
  pad = per_tile * NW - e

  src = edge_index[0].astype(jnp.int32)
  dst = edge_index[1].astype(jnp.int32)
  w = edge_weight[:, 0]
  if pad:
    src = jnp.concatenate([src, jnp.zeros((pad,), jnp.int32)])
    dst = jnp.concatenate([dst, jnp.zeros((pad,), jnp.int32)])
    w = jnp.concatenate([w, jnp.zeros((pad,), jnp.float32)])
  src = src.reshape(NW, n_chunks, CHUNK)
  dst = dst.reshape(NW, n_chunks, CHUNK)
  w = w.reshape(NW, n_chunks, CHUNK)

  agg = _make_agg(n_chunks)
  p1 = agg(x, src, dst, w)
  h = _mm_tanh(p1, W1, b1)
  p2 = agg(h, src, dst, w)
  out = _mm_tanh_mean(p2, W2, b2)
  return out.reshape(D)
